# Initial kernel scaffold; baseline (speedup 1.0000x reference)
import jax
import jax.numpy as jnp
from jax.experimental import pallas as pl


def kernel(node_features, edge_index, W, a_src, a_dst, out_w, out_b):
    raise NotImplementedError("write your pallas kernel here")


# speedup vs baseline: 82.4886x; 82.4886x over previous
import functools

import jax
import jax.numpy as jnp
import numpy as np
from jax import lax
from jax.experimental import pallas as pl
from jax.experimental.pallas import tpu as pltpu
from jax.experimental.pallas import tpu_sc as plsc

N_NODES = 10000
NODE_DIM = 128
HIDDEN_DIM = 32
NUM_HEADS = 4
N_EDGES = 320000

NC = 2
NS = 16
NW = NC * NS
EDGES_PER_W = N_EDGES // NW
CHUNK = 80
N_CHUNKS = EDGES_PER_W // CHUNK
ROWS_PER_TILE = 624
S8 = 8
HW = NODE_DIM // 2


def _mm_logits_body(x_ref, wt_ref, wa_ref, hp_ref, s8_ref):
    x = x_ref[...]
    h = jnp.dot(x, wt_ref[...], preferred_element_type=jnp.float32)
    hp_ref[...] = h.astype(jnp.bfloat16)
    s8_ref[...] = jnp.dot(x, wa_ref[...], preferred_element_type=jnp.float32)


def _stage1(x, wt_perm, wa):
    return pl.pallas_call(
        _mm_logits_body,
        out_shape=(
            jax.ShapeDtypeStruct((N_NODES, NODE_DIM), jnp.bfloat16),
            jax.ShapeDtypeStruct((N_NODES, S8), jnp.float32),
        ),
    )(x, wt_perm, wa)


def _sc_body(hp_hbm, s8_hbm, row_hbm, col_hbm, agg_out, den_out,
             hb_a, hb_b, hbuf_a, hbuf_b, wv_a, wv_b,
             rowv_a, colv_a, rowv_b, colv_b, cols_a, cols_b,
             wt_v, ssrc_a, ssrc_b, sdst_a, sdst_b,
             agg_sh, den_sh, s8_sp,
             sem_a, sem_b, semh_a, semh_b, semw_a, semw_b,
             sems_a, sems_b):
    cid = lax.axis_index("c")
    sid = lax.axis_index("s")
    wid = sid * NC + cid
    lane = lax.iota(jnp.int32, 16)
    zero16 = jnp.zeros((16,), jnp.float32)
    himask = jnp.full((16,), -65536, jnp.int32)

    def _zrow(i, _):
        for k in range(NODE_DIM // 16):
            hbuf_a[i, pl.ds(k * 16, 16)] = zero16
        return 0
    lax.fori_loop(0, CHUNK, _zrow, 0)
    wt_v[pl.ds(NUM_HEADS * CHUNK, 16)] = zero16

    def _zwv(p, _):
        ridx = 2 * p + (lane >> 3)
        cidx = lane & 7
        plsc.store_scatter(wv_a, [ridx, cidx], zero16)
        plsc.store_scatter(wv_b, [ridx, cidx], zero16)
        return 0
    lax.fori_loop(0, CHUNK // 2, _zwv, 0)

    r0 = sid * ROWS_PER_TILE
    tail0 = NS * ROWS_PER_TILE

    def _zero_slice(base, n):
        n_full, rem = n // CHUNK, n % CHUNK
        for j in range(n_full):
            pltpu.sync_copy(hbuf_a, agg_sh.at[pl.ds(base + j * CHUNK, CHUNK)])
            pltpu.sync_copy(wv_a, den_sh.at[pl.ds(base + j * CHUNK, CHUNK)])
        if rem:
            pltpu.sync_copy(hbuf_a.at[pl.ds(0, rem)],
                            agg_sh.at[pl.ds(base + n_full * CHUNK, rem)])
            pltpu.sync_copy(wv_a.at[pl.ds(0, rem)],
                            den_sh.at[pl.ds(base + n_full * CHUNK, rem)])

    _zero_slice(r0, ROWS_PER_TILE)

    @pl.when(sid == NS - 1)
    def _():
        _zero_slice(tail0, N_NODES - tail0)

    @pl.when(sid == 0)
    def _():
        pltpu.sync_copy(s8_hbm, s8_sp)

    plsc.subcore_barrier()

    ebase = wid * EDGES_PER_W

    def _issue_idx(ci, R, C, sem):
        base = ebase + ci * CHUNK
        pltpu.async_copy(row_hbm.at[pl.ds(base, CHUNK)], R, sem)
        pltpu.async_copy(col_hbm.at[pl.ds(base, CHUNK)], C, sem)

    def _wait_idx(R, C, sem):
        pltpu.make_async_copy(row_hbm.at[pl.ds(0, CHUNK)], R, sem).wait()
        pltpu.make_async_copy(col_hbm.at[pl.ds(0, CHUNK)], C, sem).wait()

    def _issue_gathers(R, C, HB, SS, SD, semh, sems):
        pltpu.async_copy(s8_sp.at[R], SS, sems)
        pltpu.async_copy(s8_sp.at[C], SD, sems)
        pltpu.async_copy(hp_hbm.at[R], HB, semh)

    def _wait_scatters(HBUF, WV, semw):
        pltpu.make_async_copy(HBUF, agg_sh.at[pl.ds(0, CHUNK)], semw).wait()
        pltpu.make_async_copy(WV, den_sh.at[pl.ds(0, CHUNK)], semw).wait()

    def _phase(ci, HB, HBUF, WV, C, CS, SS, SD, semh, sems, semw):
        pltpu.make_async_copy(s8_sp.at[C], SS, sems).wait()
        pltpu.make_async_copy(s8_sp.at[C], SD, sems).wait()
        for g in range(CHUNK // 16):
            eid = lane + g * 16
            for hh in range(NUM_HEADS):
                hv = jnp.full((16,), hh, jnp.int32)
                es = plsc.load_gather(SS, [eid, hv])
                ed = plsc.load_gather(SD, [eid, hv + NUM_HEADS])
                e = es + ed
                e = jnp.where(e >= 0.0, e, 0.2 * e)
                w = jnp.exp(e)
                wt_v[pl.ds(hh * CHUNK + g * 16, 16)] = w
                plsc.store_scatter(WV, [eid, hv], w)
        for g in range(CHUNK // 16):
            CS[pl.ds(g * 16, 16)] = C[pl.ds(g * 16, 16)]
        pltpu.make_async_copy(hp_hbm.at[C], HB, semh).wait()

        def _edge(i, _):
            idxv = jnp.where(lane < NUM_HEADS, lane * CHUNK + i,
                             NUM_HEADS * CHUNK)
            wrow = plsc.load_gather(wt_v, [idxv])
            for hh in range(NUM_HEADS):
                a = wrow[hh]
                v = HB[i, pl.ds(hh * 16, 16)]
                lo = plsc.bitcast(v << 16, jnp.float32)
                hi = plsc.bitcast(v & himask, jnp.float32)
                HBUF[i, pl.ds(hh * HIDDEN_DIM, 16)] = lo * a
                HBUF[i, pl.ds(hh * HIDDEN_DIM + 16, 16)] = hi * a
            return 0
        lax.fori_loop(0, CHUNK, _edge, 0, unroll=4)

        pltpu.async_copy(HBUF, agg_sh.at[CS], semw, add=True)
        pltpu.async_copy(WV, den_sh.at[CS], semw, add=True)

    def _tail(ci, R, C, sem, HBn, Rn, Cn, SSn, SDn, HBUFn, WVn,
              semn, semhn, semwn, semsn, wait_other_scatter):
        @pl.when(ci + 2 < N_CHUNKS)
        def _():
            _issue_idx(ci + 2, R, C, sem)
        _wait_idx(Rn, Cn, semn)
        if wait_other_scatter:
            _wait_scatters(HBUFn, WVn, semwn)
        _issue_gathers(Rn, Cn, HBn, SSn, SDn, semhn, semsn)

    _issue_idx(0, rowv_a, colv_a, sem_a)
    _issue_idx(1, rowv_b, colv_b, sem_b)
    _wait_idx(rowv_a, colv_a, sem_a)
    _issue_gathers(rowv_a, colv_a, hb_a, ssrc_a, sdst_a, semh_a, sems_a)

    A = (hb_a, hbuf_a, wv_a, rowv_a, colv_a, cols_a, ssrc_a, sdst_a,
         sem_a, semh_a, semw_a, sems_a)
    B = (hb_b, hbuf_b, wv_b, rowv_b, colv_b, cols_b, ssrc_b, sdst_b,
         sem_b, semh_b, semw_b, sems_b)

    def _step(ci, S, Sn, wait_other_scatter, steady):
        HB, HBUF, WV, R, C, CS, SS, SD, sem, semh, semw, sems = S
        (HBn, HBUFn, WVn, Rn, Cn, CSn, SSn, SDn,
         semn, semhn, semwn, semsn) = Sn
        _phase(ci, HB, HBUF, WV, C, CS, SS, SD, semh, sems, semw)
        if steady:
            _tail(ci, R, C, sem, HBn, Rn, Cn, SSn, SDn, HBUFn, WVn,
                  semn, semhn, semwn, semsn, wait_other_scatter)

    _step(0, A, B, False, True)
    _step(1, B, A, True, True)

    def _pair(j, _):
        _step(2 * j, A, B, True, True)
        _step(2 * j + 1, B, A, True, True)
        return 0

    lax.fori_loop(1, (N_CHUNKS - 1) // 2, _pair, 0)
    _step(N_CHUNKS - 1, A, B, False, False)
    _wait_scatters(hbuf_b, wv_b, semw_b)
    _wait_scatters(hbuf_a, wv_a, semw_a)

    plsc.subcore_barrier()

    def _copy_out(base, n):
        obase = cid * N_NODES + base
        pltpu.sync_copy(agg_sh.at[pl.ds(base, n)],
                        agg_out.at[pl.ds(obase, n)])
        pltpu.sync_copy(den_sh.at[pl.ds(base, n)],
                        den_out.at[pl.ds(obase, n)])

    _copy_out(r0, ROWS_PER_TILE)

    @pl.when(sid == NS - 1)
    def _():
        _copy_out(tail0, N_NODES - tail0)


def _stage2(hp_i32, s8, row, col):
    mesh = plsc.VectorSubcoreMesh(
        core_axis_name="c", subcore_axis_name="s",
        num_cores=NC, num_subcores=NS)
    f = pl.kernel(
        _sc_body,
        out_type=(
            jax.ShapeDtypeStruct((NC * N_NODES, NODE_DIM), jnp.float32),
            jax.ShapeDtypeStruct((NC * N_NODES, S8), jnp.float32),
        ),
        mesh=mesh,
        compiler_params=pltpu.CompilerParams(
            needs_layout_passes=False, use_tc_tiling_on_sc=False),
        scratch_types=(
            pltpu.VMEM((CHUNK, HW), jnp.int32),
            pltpu.VMEM((CHUNK, HW), jnp.int32),
            pltpu.VMEM((CHUNK, NODE_DIM), jnp.float32),
            pltpu.VMEM((CHUNK, NODE_DIM), jnp.float32),
            pltpu.VMEM((CHUNK, S8), jnp.float32),
            pltpu.VMEM((CHUNK, S8), jnp.float32),
            pltpu.VMEM((CHUNK,), jnp.int32),
            pltpu.VMEM((CHUNK,), jnp.int32),
            pltpu.VMEM((CHUNK,), jnp.int32),
            pltpu.VMEM((CHUNK,), jnp.int32),
            pltpu.VMEM((CHUNK,), jnp.int32),
            pltpu.VMEM((CHUNK,), jnp.int32),
            pltpu.VMEM((NUM_HEADS * CHUNK + 16,), jnp.float32),
            pltpu.VMEM((CHUNK, S8), jnp.float32),
            pltpu.VMEM((CHUNK, S8), jnp.float32),
            pltpu.VMEM((CHUNK, S8), jnp.float32),
            pltpu.VMEM((CHUNK, S8), jnp.float32),
            pltpu.VMEM_SHARED((N_NODES, NODE_DIM), jnp.float32),
            pltpu.VMEM_SHARED((N_NODES, S8), jnp.float32),
            pltpu.VMEM_SHARED((N_NODES, S8), jnp.float32),
            pltpu.SemaphoreType.DMA,
            pltpu.SemaphoreType.DMA,
            pltpu.SemaphoreType.DMA,
            pltpu.SemaphoreType.DMA,
            pltpu.SemaphoreType.DMA,
            pltpu.SemaphoreType.DMA,
            pltpu.SemaphoreType.DMA,
            pltpu.SemaphoreType.DMA,
        ),
    )
    return f(hp_i32, s8, row, col)


def _final_body(x_ref, agg_ref, den_ref, r8_ref, wo_ref, b_ref, o_ref):
    a = agg_ref[0] + agg_ref[1]
    d = den_ref[0] + den_ref[1]
    rep = jnp.dot(d, r8_ref[...], preferred_element_type=jnp.float32)
    norm = a / (rep + 1e-8)
    o_ref[...] = (x_ref[...] + b_ref[...]
                  + jnp.dot(norm, wo_ref[...],
                            preferred_element_type=jnp.float32))


def _stage3(x, agg, den, r8, wo_t, b):
    return pl.pallas_call(
        _final_body,
        out_shape=jax.ShapeDtypeStruct((N_NODES, NODE_DIM), jnp.float32),
    )(x, agg, den, r8, wo_t, b)


_PERM = np.zeros((NODE_DIM,), np.int32)
for _h in range(NUM_HEADS):
    _base = _h * HIDDEN_DIM
    for _j in range(16):
        _PERM[_base + 2 * _j] = _base + _j
        _PERM[_base + 2 * _j + 1] = _base + 16 + _j


@jax.jit
def _run(node_features, edge_index, W, a_src, a_dst, out_w, out_b):
    x = node_features
    row = edge_index[0].astype(jnp.int32)
    col = edge_index[1].astype(jnp.int32)

    wt = W.T
    wt_perm = wt[:, _PERM]
    A = jnp.zeros((NODE_DIM, S8), jnp.float32)
    for hh in range(NUM_HEADS):
        sl = slice(hh * HIDDEN_DIM, (hh + 1) * HIDDEN_DIM)
        A = A.at[sl, hh].set(a_src[hh])
        A = A.at[sl, NUM_HEADS + hh].set(a_dst[hh])
    wa = wt @ A
    r8 = jnp.zeros((S8, NODE_DIM), jnp.float32)
    for hh in range(NUM_HEADS):
        sl = slice(hh * HIDDEN_DIM, (hh + 1) * HIDDEN_DIM)
        r8 = r8.at[hh, sl].set(1.0)
    wo_t = out_w.T
    b = out_b.reshape(1, NODE_DIM)

    hp_bf, s8 = _stage1(x, wt_perm, wa)
    hp_i32 = lax.bitcast_convert_type(
        hp_bf.reshape(N_NODES, HW, 2), jnp.int32)
    agg2, den2 = _stage2(hp_i32, s8, row, col)
    agg = agg2.reshape(NC, N_NODES, NODE_DIM)
    den = den2.reshape(NC, N_NODES, S8)
    return _stage3(x, agg, den, r8, wo_t, b)


def kernel(node_features, edge_index, W, a_src, a_dst, out_w, out_b):
    return _run(node_features, edge_index, W, a_src, a_dst, out_w, out_b)
